# Initial kernel scaffold; baseline (speedup 1.0000x reference)
import jax
import jax.numpy as jnp
from jax.experimental import pallas as pl


def kernel(x, table, W, b):
    raise NotImplementedError("write your pallas kernel here")


# speedup vs baseline: 3.8884x; 3.8884x over previous
import jax
import jax.numpy as jnp
from jax import lax
from jax.experimental import pallas as pl
from jax.experimental.pallas import tpu as pltpu
from jax.experimental.pallas import tpu_sc as plsc

_RANK = 32
_DIM = 128
_NC = 2
_NS = 16
_NW = _NC * _NS
_CH = 128


def _gather_body(table_hbm, idx_hbm, out_hbm, idx_v, rows_v, sem):
    nchunk = idx_v.shape[0]
    wid = lax.axis_index("s") * _NC + lax.axis_index("c")
    pltpu.sync_copy(idx_hbm.at[wid], idx_v)
    copies = [
        pltpu.async_copy(table_hbm.at[idx_v.at[j]], rows_v.at[j], sem)
        for j in range(nchunk)
    ]
    for c in copies:
        c.wait()
    pltpu.sync_copy(rows_v, out_hbm.at[wid])


def _sc_gather(table, idx3):
    nchunk = idx3.shape[1]
    fn = pl.kernel(
        _gather_body,
        out_type=jax.ShapeDtypeStruct((_NW, nchunk, _CH, _RANK), jnp.float32),
        mesh=plsc.VectorSubcoreMesh(core_axis_name="c", subcore_axis_name="s"),
        scratch_types=[
            pltpu.VMEM((nchunk, _CH), jnp.int32),
            pltpu.VMEM((nchunk, _CH, _RANK), jnp.float32),
            pltpu.SemaphoreType.DMA,
        ],
        compiler_params=pltpu.CompilerParams(use_tc_tiling_on_sc=False),
    )
    return fn(table, idx3)


def _mm_body(low_ref, w_ref, b_ref, out_ref):
    out_ref[...] = (
        jnp.dot(low_ref[...], w_ref[...], preferred_element_type=jnp.float32)
        + b_ref[...]
    )


def _tc_project(low, W, b, bm):
    n = low.shape[0]
    return pl.pallas_call(
        _mm_body,
        grid=(n // bm,),
        in_specs=[
            pl.BlockSpec((bm, _RANK), lambda i: (i, 0)),
            pl.BlockSpec((_RANK, _DIM), lambda i: (0, 0)),
            pl.BlockSpec((1, _DIM), lambda i: (0, 0)),
        ],
        out_specs=pl.BlockSpec((bm, _DIM), lambda i: (i, 0)),
        out_shape=jax.ShapeDtypeStruct((n, _DIM), jnp.float32),
    )(low, W, b.reshape(1, _DIM))


def kernel(x, table, W, b):
    bsz, f = x.shape
    n = bsz * f
    nchunk = n // (_NW * _CH)
    idx3 = x.reshape(_NW, nchunk, _CH).astype(jnp.int32)
    low = _sc_gather(table, idx3).reshape(n, _RANK)
    out = _tc_project(low, W, b, bm=2048)
    return out.reshape(bsz, f, _DIM)


# speedup vs baseline: 4.7206x; 1.2140x over previous
import jax
import jax.numpy as jnp
from jax import lax
from jax.experimental import pallas as pl
from jax.experimental.pallas import tpu as pltpu
from jax.experimental.pallas import tpu_sc as plsc

_RANK = 32
_DIM = 128
_NC = 2
_NS = 16
_NW = _NC * _NS
_CH = 128


_DEPTH = 4


def _gather_body(table_hbm, idx_hbm, out_hbm, idx_v, rows_v, gsem, osem):
    nchunk = idx_v.shape[0]
    wid = lax.axis_index("s") * _NC + lax.axis_index("c")
    pltpu.sync_copy(idx_hbm.at[wid], idx_v)
    gc = [None] * nchunk
    oc = [None] * nchunk
    for j in range(nchunk):
        if j >= _DEPTH:
            k = j - _DEPTH
            gc[k].wait()
            oc[k] = pltpu.async_copy(
                rows_v.at[k % _DEPTH], out_hbm.at[wid, k], osem.at[k % _DEPTH]
            )
            oc[k].wait()
        gc[j] = pltpu.async_copy(
            table_hbm.at[idx_v.at[j]], rows_v.at[j % _DEPTH],
            gsem.at[j % _DEPTH],
        )
    for k in range(nchunk - _DEPTH, nchunk):
        gc[k].wait()
        oc[k] = pltpu.async_copy(
            rows_v.at[k % _DEPTH], out_hbm.at[wid, k], osem.at[k % _DEPTH]
        )
    for k in range(nchunk - _DEPTH, nchunk):
        oc[k].wait()


def _sc_gather(table, idx3):
    nchunk = idx3.shape[1]
    fn = pl.kernel(
        _gather_body,
        out_type=jax.ShapeDtypeStruct((_NW, nchunk, _CH, _DIM), jnp.float32),
        mesh=plsc.VectorSubcoreMesh(core_axis_name="c", subcore_axis_name="s"),
        scratch_types=[
            pltpu.VMEM((nchunk, _CH), jnp.int32),
            pltpu.VMEM((_DEPTH, _CH, _DIM), jnp.float32),
            pltpu.SemaphoreType.DMA((_DEPTH,)),
            pltpu.SemaphoreType.DMA((_DEPTH,)),
        ],
        compiler_params=pltpu.CompilerParams(use_tc_tiling_on_sc=False),
    )
    return fn(table, idx3)


def _mm_body(low_ref, w_ref, b_ref, out_ref):
    out_ref[...] = (
        jnp.dot(low_ref[:, :_RANK], w_ref[...],
                preferred_element_type=jnp.float32)
        + b_ref[...]
    )


def _tc_project(low128, W, b, bm):
    n = low128.shape[0]
    return pl.pallas_call(
        _mm_body,
        grid=(n // bm,),
        in_specs=[
            pl.BlockSpec((bm, _DIM), lambda i: (i, 0)),
            pl.BlockSpec((_RANK, _DIM), lambda i: (0, 0)),
            pl.BlockSpec((1, _DIM), lambda i: (0, 0)),
        ],
        out_specs=pl.BlockSpec((bm, _DIM), lambda i: (i, 0)),
        out_shape=jax.ShapeDtypeStruct((n, _DIM), jnp.float32),
    )(low128, W, b.reshape(1, _DIM))


def kernel(x, table, W, b):
    bsz, f = x.shape
    n = bsz * f
    nchunk = n // (_NW * _CH)
    tabp = jnp.pad(table, ((0, 0), (0, _DIM - _RANK)))
    idx3 = x.T.reshape(_NW, nchunk, _CH).astype(jnp.int32)
    low = _sc_gather(tabp, idx3)
    low128 = low.reshape(n, _DIM)
    out = _tc_project(low128, W, b, bm=2048)
    return out.reshape(f, bsz, _DIM).transpose(1, 0, 2)
